# Initial kernel scaffold; baseline (speedup 1.0000x reference)
import jax
import jax.numpy as jnp
from jax.experimental import pallas as pl


def kernel(logits, z_t):
    raise NotImplementedError("write your pallas kernel here")


# speedup vs baseline: 1.7218x; 1.7218x over previous
import jax
import jax.numpy as jnp
from jax.experimental import pallas as pl
from jax.experimental.pallas import tpu as pltpu

VOCAB = 32768
MASK_ID = 32767
ROWS_PER_BLOCK = 16


def _blend_kernel(z_ref, logits_ref, out_ref):
    z = z_ref[:, :]
    r, c = out_ref.shape
    col = jax.lax.broadcasted_iota(jnp.int32, (r, c), 1)
    onehot = jnp.where(col == z, jnp.float32(1e9), jnp.float32(0.0))
    masked = z == MASK_ID
    lg = logits_ref[:, :]
    lg = jnp.where(col == MASK_ID, jnp.float32(-jnp.inf), lg)
    out_ref[:, :] = jnp.where(masked, lg, onehot)


def kernel(logits, z_t):
    b, t, v = logits.shape
    n = b * t
    lf = logits.reshape(n, v)
    zf = z_t.reshape(n, 1)
    r = ROWS_PER_BLOCK
    out = pl.pallas_call(
        _blend_kernel,
        grid=(n // r,),
        in_specs=[
            pl.BlockSpec((r, 1), lambda i: (i, 0)),
            pl.BlockSpec((r, v), lambda i: (i, 0)),
        ],
        out_specs=pl.BlockSpec((r, v), lambda i: (i, 0)),
        out_shape=jax.ShapeDtypeStruct((n, v), jnp.float32),
    )(zf, lf)
    return out.reshape(b, t, v)


# speedup vs baseline: 2.7153x; 1.5770x over previous
import jax
import jax.numpy as jnp
from jax.experimental import pallas as pl
from jax.experimental.pallas import tpu as pltpu

VOCAB = 32768
MASK_ID = 32767
ROWS_PER_BLOCK = 16


def _blend_kernel(z_ref, logits_hbm, out_ref, scratch, sem):
    i = pl.program_id(0)
    z = z_ref[:, :]
    r, c = out_ref.shape
    col = jax.lax.broadcasted_iota(jnp.int32, (r, c), 1)
    onehot = jnp.where(col == z, jnp.float32(1e9), jnp.float32(0.0))
    any_masked = jnp.any(z == MASK_ID)

    @pl.when(any_masked)
    def _():
        cp = pltpu.make_async_copy(
            logits_hbm.at[pl.ds(i * r, r), :], scratch, sem
        )
        cp.start()
        cp.wait()
        lg = jnp.where(col == MASK_ID, jnp.float32(-jnp.inf), scratch[:, :])
        out_ref[:, :] = jnp.where(z == MASK_ID, lg, onehot)

    @pl.when(jnp.logical_not(any_masked))
    def _():
        out_ref[:, :] = onehot


def kernel(logits, z_t):
    b, t, v = logits.shape
    n = b * t
    lf = logits.reshape(n, v)
    zf = z_t.reshape(n, 1)
    r = ROWS_PER_BLOCK
    out = pl.pallas_call(
        _blend_kernel,
        grid=(n // r,),
        in_specs=[
            pl.BlockSpec((r, 1), lambda i: (i, 0)),
            pl.BlockSpec(memory_space=pl.ANY),
        ],
        out_specs=pl.BlockSpec((r, v), lambda i: (i, 0)),
        out_shape=jax.ShapeDtypeStruct((n, v), jnp.float32),
        scratch_shapes=[
            pltpu.VMEM((r, v), jnp.float32),
            pltpu.SemaphoreType.DMA,
        ],
    )(zf, lf)
    return out.reshape(b, t, v)


# speedup vs baseline: 3.7678x; 1.3876x over previous
import jax
import jax.numpy as jnp
from jax.experimental import pallas as pl
from jax.experimental.pallas import tpu as pltpu

VOCAB = 32768
MASK_ID = 32767
ROWS_PER_BLOCK = 32


def _blend_kernel(z_ref, logits_hbm, out_ref, scratch, sem):
    i = pl.program_id(0)
    z = z_ref[:, :]
    r, c = out_ref.shape
    col = jax.lax.broadcasted_iota(jnp.int32, (r, c), 1)
    onehot = jnp.where(col == z, jnp.float32(1e9), jnp.float32(0.0))
    any_masked = jnp.any(z == MASK_ID)

    @pl.when(any_masked)
    def _():
        cp = pltpu.make_async_copy(
            logits_hbm.at[pl.ds(i * r, r), :], scratch, sem
        )
        cp.start()
        cp.wait()
        lg = jnp.where(col == MASK_ID, jnp.float32(-jnp.inf), scratch[:, :])
        out_ref[:, :] = jnp.where(z == MASK_ID, lg, onehot)

    @pl.when(jnp.logical_not(any_masked))
    def _():
        out_ref[:, :] = onehot


def kernel(logits, z_t):
    b, t, v = logits.shape
    n = b * t
    lf = logits.reshape(n, v)
    zf = z_t.reshape(n, 1)
    r = ROWS_PER_BLOCK
    out = pl.pallas_call(
        _blend_kernel,
        grid=(n // r,),
        in_specs=[
            pl.BlockSpec((r, 1), lambda i: (i, 0)),
            pl.BlockSpec(memory_space=pl.ANY),
        ],
        out_specs=pl.BlockSpec((r, v), lambda i: (i, 0)),
        out_shape=jax.ShapeDtypeStruct((n, v), jnp.float32),
        scratch_shapes=[
            pltpu.VMEM((r, v), jnp.float32),
            pltpu.SemaphoreType.DMA,
        ],
    )(zf, lf)
    return out.reshape(b, t, v)


# speedup vs baseline: 4.1438x; 1.0998x over previous
import jax
import jax.numpy as jnp
from jax.experimental import pallas as pl
from jax.experimental.pallas import tpu as pltpu

VOCAB = 32768
MASK_ID = 32767
ROWS_PER_BLOCK = 64


def _blend_kernel(z_ref, logits_hbm, out_ref, scratch, sem):
    i = pl.program_id(0)
    z = z_ref[:, :]
    r, c = out_ref.shape
    col = jax.lax.broadcasted_iota(jnp.int32, (r, c), 1)
    onehot = jnp.where(col == z, jnp.float32(1e9), jnp.float32(0.0))
    any_masked = jnp.any(z == MASK_ID)

    @pl.when(any_masked)
    def _():
        cp = pltpu.make_async_copy(
            logits_hbm.at[pl.ds(i * r, r), :], scratch, sem
        )
        cp.start()
        cp.wait()
        lg = jnp.where(col == MASK_ID, jnp.float32(-jnp.inf), scratch[:, :])
        out_ref[:, :] = jnp.where(z == MASK_ID, lg, onehot)

    @pl.when(jnp.logical_not(any_masked))
    def _():
        out_ref[:, :] = onehot


def kernel(logits, z_t):
    b, t, v = logits.shape
    n = b * t
    lf = logits.reshape(n, v)
    zf = z_t.reshape(n, 1)
    r = ROWS_PER_BLOCK
    out = pl.pallas_call(
        _blend_kernel,
        grid=(n // r,),
        in_specs=[
            pl.BlockSpec((r, 1), lambda i: (i, 0)),
            pl.BlockSpec(memory_space=pl.ANY),
        ],
        out_specs=pl.BlockSpec((r, v), lambda i: (i, 0)),
        out_shape=jax.ShapeDtypeStruct((n, v), jnp.float32),
        scratch_shapes=[
            pltpu.VMEM((r, v), jnp.float32),
            pltpu.SemaphoreType.DMA,
        ],
    )(zf, lf)
    return out.reshape(b, t, v)
